# baseline (speedup 1.0000x reference)
import functools

import jax
import jax.numpy as jnp
from jax import lax
from jax.experimental import pallas as pl
from jax.experimental.pallas import tpu as pltpu
from jax.experimental.pallas import tpu_sc as plsc


def kernel(output, length):
    B, T, D = output.shape
    flat = output.reshape(B * T, D)
    mesh = plsc.VectorSubcoreMesh(core_axis_name="c", subcore_axis_name="s")

    @functools.partial(
        pl.kernel,
        mesh=mesh,
        out_type=jax.ShapeDtypeStruct((B, D), jnp.float32),
        scratch_types=[
            pltpu.VMEM((B,), jnp.int32),
            pltpu.VMEM((B, D), jnp.float32),
            pltpu.SemaphoreType.DMA,
        ],
    )
    def gather_last(len_hbm, flat_hbm, out_hbm, idx_v, rows_v, sem):
        cid = lax.axis_index("c")
        sid = lax.axis_index("s")

        @pl.when(jnp.logical_and(cid == 0, sid == 0))
        def _():
            pltpu.sync_copy(len_hbm, idx_v)
            idx_v[...] = lax.iota(jnp.int32, B) * T + idx_v[...] - 1
            pltpu.async_copy(flat_hbm.at[idx_v], rows_v, sem).wait()
            pltpu.sync_copy(rows_v, out_hbm)

    return gather_last(length, flat)


# speedup vs baseline: 1.0355x; 1.0355x over previous
import functools

import jax
import jax.numpy as jnp
from jax import lax
from jax.experimental import pallas as pl
from jax.experimental.pallas import tpu as pltpu
from jax.experimental.pallas import tpu_sc as plsc


def kernel(output, length):
    B, T, D = output.shape
    mesh = plsc.VectorSubcoreMesh(core_axis_name="c", subcore_axis_name="s")

    @functools.partial(
        pl.kernel,
        mesh=mesh,
        out_type=jax.ShapeDtypeStruct((B, D), jnp.float32),
        scratch_types=[
            pltpu.VMEM((B,), jnp.int32),
            pltpu.VMEM((1, D), jnp.float32),
            pltpu.SemaphoreType.DMA,
        ],
    )
    def gather_last(len_hbm, seq_hbm, out_hbm, idx_v, row_v, sem):
        cid = lax.axis_index("c")
        sid = lax.axis_index("s")
        wid = cid * 16 + sid

        @pl.when(wid < B)
        def _():
            pltpu.sync_copy(len_hbm, idx_v)
            vals = idx_v[...] - 1
            pos = jnp.full((B, 1), wid, dtype=jnp.int32)
            idx_v[...] = lax.gather(
                vals,
                pos,
                lax.GatherDimensionNumbers(
                    offset_dims=(),
                    collapsed_slice_dims=(0,),
                    start_index_map=(0,),
                ),
                slice_sizes=(1,),
                mode=lax.GatherScatterMode.PROMISE_IN_BOUNDS,
            )
            pltpu.async_copy(
                seq_hbm.at[wid].at[idx_v.at[pl.ds(0, 1)]], row_v, sem
            ).wait()
            pltpu.sync_copy(row_v, out_hbm.at[pl.ds(wid, 1)])

    return gather_last(length, output)


# speedup vs baseline: 1.1090x; 1.0710x over previous
import functools

import jax
import jax.numpy as jnp
from jax import lax
from jax.experimental import pallas as pl
from jax.experimental.pallas import tpu as pltpu
from jax.experimental.pallas import tpu_sc as plsc


def kernel(output, length):
    B, T, D = output.shape
    mesh = plsc.ScalarSubcoreMesh(axis_name="c", num_cores=1)

    @functools.partial(
        pl.kernel,
        mesh=mesh,
        out_type=jax.ShapeDtypeStruct((B, D), jnp.float32),
        scratch_types=[
            pltpu.SMEM((B,), jnp.int32),
            pltpu.SemaphoreType.DMA,
        ],
    )
    def gather_last(len_hbm, seq_hbm, out_hbm, len_s, sem):
        pltpu.sync_copy(len_hbm, len_s)
        copies = []
        for b in range(B):
            t = len_s[b] - 1
            copies.append(
                pltpu.make_async_copy(
                    seq_hbm.at[b].at[pl.ds(t, 1)], out_hbm.at[pl.ds(b, 1)], sem
                )
            )
        for c in copies:
            c.start()
        for c in copies:
            c.wait()

    return gather_last(length, output)
